# baseline (speedup 1.0000x reference)
import functools

import jax
import jax.numpy as jnp
from jax import lax
from jax.experimental import pallas as pl
from jax.experimental.pallas import tpu as pltpu
from jax.experimental.pallas import tpu_sc as plsc

MARGIN = 1.0

_CHUNK = 4000
_NBUF = 8


def _scores_body(n_chunks, e_ref, u_ref, o_ref, bufs, sems):
    def start(i, slot):
        pltpu.make_async_copy(
            e_ref.at[pl.ds(i * _CHUNK, _CHUNK)], bufs.at[slot], sems.at[slot]
        ).start()

    for i in range(_NBUF):
        start(i, i)

    def step(i, carry):
        slot = lax.rem(i, _NBUF)
        pltpu.make_async_copy(
            e_ref.at[pl.ds(i * _CHUNK, _CHUNK)], bufs.at[slot], sems.at[slot]
        ).wait()
        res = lax.dot_general(
            u_ref[...], bufs[slot],
            dimension_numbers=(((1,), (1,)), ((), ())),
            preferred_element_type=jnp.float32,
        )
        o_ref[pl.ds(i, 1), :] = res

        nxt = i + _NBUF

        @pl.when(nxt < n_chunks)
        def _():
            start(nxt, slot)

        return carry

    lax.fori_loop(0, n_chunks, step, 0)


def _compute_scores(E, U):
    V, D = E.shape
    n_chunks = V // _CHUNK
    assert n_chunks * _CHUNK == V
    out = pl.pallas_call(
        functools.partial(_scores_body, n_chunks),
        in_specs=[
            pl.BlockSpec(memory_space=pl.ANY),
            pl.BlockSpec(memory_space=pltpu.VMEM),
        ],
        out_specs=pl.BlockSpec(memory_space=pltpu.VMEM),
        out_shape=jax.ShapeDtypeStruct((n_chunks, _CHUNK), jnp.float32),
        scratch_shapes=[
            pltpu.VMEM((_NBUF, _CHUNK, D), jnp.float32),
            pltpu.SemaphoreType.DMA((_NBUF,)),
        ],
    )(E, U)
    return out.reshape(V)


def _make_sc_loss(n_pairs, n_workers, margin):
    per_w = n_pairs // n_workers
    n_vec = per_w // 16
    mesh = plsc.VectorSubcoreMesh(core_axis_name="c", subcore_axis_name="s")

    @functools.partial(
        pl.kernel,
        out_type=jax.ShapeDtypeStruct((n_workers, 16), jnp.float32),
        mesh=mesh,
        scratch_types=[
            pltpu.VMEM((per_w,), jnp.int32),
            pltpu.VMEM((per_w,), jnp.int32),
            pltpu.VMEM((per_w,), jnp.float32),
            pltpu.VMEM((per_w,), jnp.float32),
            pltpu.VMEM((16,), jnp.float32),
            pltpu.SemaphoreType.DMA,
        ],
    )
    def sc_loss(scores_hbm, doc_hbm, neg_hbm, out_hbm,
                idx_d, idx_n, sd, sn, accv, sem):
        wid = lax.axis_index("s") * 2 + lax.axis_index("c")
        base = wid * per_w
        pltpu.sync_copy(doc_hbm.at[pl.ds(base, per_w)], idx_d)
        pltpu.sync_copy(neg_hbm.at[pl.ds(base, per_w)], idx_n)
        cp_d = pltpu.async_copy(scores_hbm.at[idx_d], sd, sem)
        cp_n = pltpu.async_copy(scores_hbm.at[idx_n], sn, sem)
        cp_d.wait()
        cp_n.wait()

        def body(i, acc):
            vd = sd[pl.ds(i * 16, 16)]
            vn = sn[pl.ds(i * 16, 16)]
            return acc + jnp.maximum(0.0, margin - (vd + vn))

        accv[...] = lax.fori_loop(0, n_vec, body,
                                  jnp.zeros((16,), jnp.float32))
        pltpu.sync_copy(accv, out_hbm.at[wid])

    return sc_loss


def kernel(E, U, doc, neg_samples):
    n_pairs = doc.shape[0] * doc.shape[1]
    scores = _compute_scores(E, U)
    sc_loss = _make_sc_loss(n_pairs, 32, MARGIN)
    partials = sc_loss(scores, doc.reshape(-1), neg_samples.reshape(-1))
    return jnp.sum(partials) / n_pairs
